# baseline (speedup 1.0000x reference)
import functools

import jax
import jax.numpy as jnp
from jax import lax
from jax.experimental import pallas as pl
from jax.experimental.pallas import tpu as pltpu
from jax.experimental.pallas import tpu_sc as plsc

_NC = 2
_NS = 16
_NW = _NC * _NS
_LANE = 128
_COPY_CHUNKS = 28
_TILE = 128


def _flatten_copy(w, aligned):
    n = _COPY_CHUNKS
    while n > 1 and (aligned // _TILE) % n:
        n -= 1
    size = aligned // n

    def body(in_ref, out_ref, sem):
        cps = [
            pltpu.make_async_copy(
                in_ref.at[0, pl.ds(i * size, size)],
                out_ref.at[pl.ds(i * size, size)],
                sem.at[i],
            )
            for i in range(n)
        ]
        for cp in cps:
            cp.start()
        for cp in cps:
            cp.wait()

    wt = w.T
    return pl.pallas_call(
        body,
        in_specs=[pl.BlockSpec(memory_space=pl.ANY)],
        out_specs=pl.BlockSpec(memory_space=pl.ANY),
        out_shape=jax.ShapeDtypeStruct((aligned,), jnp.float32),
        scratch_shapes=[pltpu.SemaphoreType.DMA((n,))],
    )(wt)


@functools.partial(jax.jit, static_argnums=(4,))
def _sc_gather2(idx_m, idx_t, table, tail, chunks):
    mesh = plsc.VectorSubcoreMesh(core_axis_name="c", subcore_axis_name="s")

    @functools.partial(
        pl.kernel,
        out_type=[
            jax.ShapeDtypeStruct((_NW, chunks, _LANE), jnp.float32),
            jax.ShapeDtypeStruct((_NW, chunks, _LANE), jnp.float32),
        ],
        mesh=mesh,
        scratch_types=[
            pltpu.VMEM((chunks, _LANE), jnp.int32),
            pltpu.VMEM((chunks, _LANE), jnp.int32),
            pltpu.VMEM((chunks, _LANE), jnp.float32),
            pltpu.VMEM((chunks, _LANE), jnp.float32),
            pltpu.SemaphoreType.DMA,
            pltpu.SemaphoreType.DMA((2 * chunks,)),
            pltpu.SemaphoreType.DMA,
        ],
    )
    def run(idxm_hbm, idxt_hbm, table_hbm, tail_hbm, outm_hbm, outt_hbm,
            idxm_v, idxt_v, rowsm_v, rowst_v, isem, gsems, osem):
        wid = lax.axis_index("s") * _NC + lax.axis_index("c")
        icp = pltpu.async_copy(idxm_hbm.at[wid], idxm_v, isem)
        pltpu.sync_copy(idxt_hbm.at[wid], idxt_v)
        icp.wait()
        gathers = []
        for j in range(chunks):
            gathers.append(pltpu.async_copy(
                table_hbm.at[idxm_v.at[j]], rowsm_v.at[j], gsems.at[2 * j]))
            gathers.append(pltpu.async_copy(
                tail_hbm.at[idxt_v.at[j]], rowst_v.at[j], gsems.at[2 * j + 1]))
        outs = []
        for j in range(chunks):
            gathers[2 * j].wait()
            outs.append(
                pltpu.async_copy(rowsm_v.at[j], outm_hbm.at[wid].at[j], osem))
            gathers[2 * j + 1].wait()
            outs.append(
                pltpu.async_copy(rowst_v.at[j], outt_hbm.at[wid].at[j], osem))
        for cp in outs:
            cp.wait()

    return run(idx_m, idx_t, table, tail)


def kernel(inputs, weight):
    batch = inputs.shape[0]
    v = weight.shape[0]
    aligned = (v // _TILE) * _TILE
    ntail = v - aligned
    table = _flatten_copy(weight, aligned)

    per_w = -(-batch // _NW)
    chunks = -(-per_w // _LANE)
    batch_pad = _NW * chunks * _LANE
    idx = inputs.astype(jnp.int32)
    if batch_pad != batch:
        idx = jnp.pad(idx, (0, batch_pad - batch))
    if ntail == 0:
        zeros = jnp.zeros((1,), jnp.float32)
        idx_m = idx.reshape(_NW, chunks, _LANE)
        idx_t = jnp.zeros_like(idx_m)
        out_m, _ = _sc_gather2(idx_m, idx_t, table, zeros, chunks)
        return out_m.reshape(batch_pad, 1)[:batch]

    tail = weight[aligned:, 0]
    idx_m = jnp.minimum(idx, aligned - 1).reshape(_NW, chunks, _LANE)
    idx_t = jnp.clip(idx - aligned, 0, ntail - 1).reshape(_NW, chunks, _LANE)
    out_m, out_t = _sc_gather2(idx_m, idx_t, table, tail, chunks)
    out = jnp.where(idx < aligned,
                    out_m.reshape(batch_pad), out_t.reshape(batch_pad))
    return out.reshape(batch_pad, 1)[:batch]


# speedup vs baseline: 1.8070x; 1.8070x over previous
import functools

import jax
import jax.numpy as jnp
from jax import lax
from jax.experimental import pallas as pl
from jax.experimental.pallas import tpu as pltpu
from jax.experimental.pallas import tpu_sc as plsc

_NC = 2
_NS = 16
_NW = _NC * _NS
_LANE = 128
_COPY_CHUNKS = 28
_TILE = 128


def _flatten_copy(w, aligned):
    n = _COPY_CHUNKS
    while n > 1 and (aligned // _TILE) % n:
        n -= 1
    size = aligned // n
    nbuf = 2

    def body(in_ref, out_ref, vmem, insems, outsems):
        loads = [None] * n
        stores = [None] * n

        def load(i):
            loads[i] = pltpu.make_async_copy(
                in_ref.at[0, pl.ds(i * size, size)],
                vmem.at[i % nbuf],
                insems.at[i % nbuf],
            )
            loads[i].start()

        def store(i):
            stores[i] = pltpu.make_async_copy(
                vmem.at[i % nbuf],
                out_ref.at[pl.ds(i * size, size)],
                outsems.at[i % nbuf],
            )
            stores[i].start()

        load(0)
        for i in range(n):
            if i + 1 < n:
                if i + 1 - nbuf >= 0:
                    stores[i + 1 - nbuf].wait()
                load(i + 1)
            loads[i].wait()
            store(i)
        for i in range(max(0, n - nbuf), n):
            stores[i].wait()

    wt = w.T
    return pl.pallas_call(
        body,
        in_specs=[pl.BlockSpec(memory_space=pl.ANY)],
        out_specs=pl.BlockSpec(memory_space=pl.ANY),
        out_shape=jax.ShapeDtypeStruct((aligned,), jnp.float32),
        scratch_shapes=[
            pltpu.VMEM((nbuf, size), jnp.float32),
            pltpu.SemaphoreType.DMA((nbuf,)),
            pltpu.SemaphoreType.DMA((nbuf,)),
        ],
    )(wt)


@functools.partial(jax.jit, static_argnums=(4,))
def _sc_gather2(idx_m, idx_t, table, tail, chunks):
    mesh = plsc.VectorSubcoreMesh(core_axis_name="c", subcore_axis_name="s")

    @functools.partial(
        pl.kernel,
        out_type=[
            jax.ShapeDtypeStruct((_NW, chunks, _LANE), jnp.float32),
            jax.ShapeDtypeStruct((_NW, chunks, _LANE), jnp.float32),
        ],
        mesh=mesh,
        scratch_types=[
            pltpu.VMEM((chunks, _LANE), jnp.int32),
            pltpu.VMEM((chunks, _LANE), jnp.int32),
            pltpu.VMEM((chunks, _LANE), jnp.float32),
            pltpu.VMEM((chunks, _LANE), jnp.float32),
            pltpu.SemaphoreType.DMA,
            pltpu.SemaphoreType.DMA((2 * chunks,)),
            pltpu.SemaphoreType.DMA,
        ],
    )
    def run(idxm_hbm, idxt_hbm, table_hbm, tail_hbm, outm_hbm, outt_hbm,
            idxm_v, idxt_v, rowsm_v, rowst_v, isem, gsems, osem):
        wid = lax.axis_index("s") * _NC + lax.axis_index("c")
        icp = pltpu.async_copy(idxm_hbm.at[wid], idxm_v, isem)
        pltpu.sync_copy(idxt_hbm.at[wid], idxt_v)
        icp.wait()
        gathers = []
        for j in range(chunks):
            gathers.append(pltpu.async_copy(
                table_hbm.at[idxm_v.at[j]], rowsm_v.at[j], gsems.at[2 * j]))
            gathers.append(pltpu.async_copy(
                tail_hbm.at[idxt_v.at[j]], rowst_v.at[j], gsems.at[2 * j + 1]))
        outs = []
        for j in range(chunks):
            gathers[2 * j].wait()
            outs.append(
                pltpu.async_copy(rowsm_v.at[j], outm_hbm.at[wid].at[j], osem))
            gathers[2 * j + 1].wait()
            outs.append(
                pltpu.async_copy(rowst_v.at[j], outt_hbm.at[wid].at[j], osem))
        for cp in outs:
            cp.wait()

    return run(idx_m, idx_t, table, tail)


def kernel(inputs, weight):
    batch = inputs.shape[0]
    v = weight.shape[0]
    aligned = (v // _TILE) * _TILE
    ntail = v - aligned
    table = _flatten_copy(weight, aligned)

    per_w = -(-batch // _NW)
    chunks = -(-per_w // _LANE)
    batch_pad = _NW * chunks * _LANE
    idx = inputs.astype(jnp.int32)
    if batch_pad != batch:
        idx = jnp.pad(idx, (0, batch_pad - batch))
    if ntail == 0:
        zeros = jnp.zeros((1,), jnp.float32)
        idx_m = idx.reshape(_NW, chunks, _LANE)
        idx_t = jnp.zeros_like(idx_m)
        out_m, _ = _sc_gather2(idx_m, idx_t, table, zeros, chunks)
        return out_m.reshape(batch_pad, 1)[:batch]

    tail = weight[aligned:, 0]
    idx_m = jnp.minimum(idx, aligned - 1).reshape(_NW, chunks, _LANE)
    idx_t = jnp.clip(idx - aligned, 0, ntail - 1).reshape(_NW, chunks, _LANE)
    out_m, out_t = _sc_gather2(idx_m, idx_t, table, tail, chunks)
    out = jnp.where(idx < aligned,
                    out_m.reshape(batch_pad), out_t.reshape(batch_pad))
    return out.reshape(batch_pad, 1)[:batch]


# speedup vs baseline: 3.5659x; 1.9735x over previous
import functools

import jax
import jax.numpy as jnp
from jax import lax
from jax.experimental import pallas as pl
from jax.experimental.pallas import tpu as pltpu
from jax.experimental.pallas import tpu_sc as plsc

_NC = 2
_NS = 16
_NW = _NC * _NS
_LANE = 128


@functools.partial(jax.jit, static_argnums=(2, 3))
def _sc_gather(idx, table, chunks, batch_pad):
    mesh = plsc.VectorSubcoreMesh(core_axis_name="c", subcore_axis_name="s")

    @functools.partial(
        pl.kernel,
        out_type=jax.ShapeDtypeStruct((_NW, chunks, _LANE), jnp.float32),
        mesh=mesh,
        scratch_types=[
            pltpu.VMEM((chunks, _LANE), jnp.int32),
            pltpu.VMEM((chunks, _LANE), jnp.float32),
            pltpu.SemaphoreType.DMA((chunks,)),
            pltpu.SemaphoreType.DMA,
        ],
    )
    def run(idx_hbm, table_hbm, out_hbm, idx_v, rows_v, gsems, osem):
        wid = lax.axis_index("s") * _NC + lax.axis_index("c")
        pltpu.sync_copy(idx_hbm.at[wid], idx_v)
        gathers = [
            pltpu.async_copy(table_hbm.at[idx_v.at[j]], rows_v.at[j], gsems.at[j])
            for j in range(chunks)
        ]
        outs = []
        for j in range(chunks):
            gathers[j].wait()
            outs.append(pltpu.async_copy(rows_v.at[j], out_hbm.at[wid].at[j], osem))
        for cp in outs:
            cp.wait()

    return run(idx, table)


def kernel(inputs, weight):
    batch = inputs.shape[0]
    table = weight.reshape(-1)
    per_w = -(-batch // _NW)
    chunks = -(-per_w // _LANE)
    batch_pad = _NW * chunks * _LANE
    idx = inputs.astype(jnp.int32)
    if batch_pad != batch:
        idx = jnp.pad(idx, (0, batch_pad - batch))
    idx = idx.reshape(_NW, chunks, _LANE)
    out = _sc_gather(idx, table, chunks, batch_pad)
    return out.reshape(batch_pad, 1)[:batch]
